# Initial kernel scaffold; baseline (speedup 1.0000x reference)
import jax
import jax.numpy as jnp
from jax.experimental import pallas as pl


def kernel(logit, target):
    raise NotImplementedError("write your pallas kernel here")


# speedup vs baseline: 1.8272x; 1.8272x over previous
import jax
import jax.numpy as jnp
from jax.experimental import pallas as pl
from jax.experimental.pallas import tpu as pltpu

_C = 100000
_IGNORE = 0
_SMOOTH = 0.1
_CONF = 1.0 - _SMOOTH
_EPS = _SMOOTH / (_C - 1)
_BV = 2048
_NBLK = (_C + _BV - 1) // _BV


def _body(tgt_ref, logit_ref, out_ref):
    j = pl.program_id(0)
    blk = logit_ref[...]
    t = tgt_ref[...]
    col = jax.lax.broadcasted_iota(jnp.int32, blk.shape, 1) + j * _BV
    row_ok = t != _IGNORE
    col_ok = col < _C
    w = jnp.where(col == t, _CONF, _EPS)
    w = jnp.where(row_ok & col_ok, w, 0.0)
    vals = jnp.where(col_ok, blk, 0.0)
    partial = jnp.sum(w * vals).reshape(1, 1)

    @pl.when(j == 0)
    def _init():
        out_ref[...] = jnp.zeros((1, 1), jnp.float32)

    out_ref[...] += -partial


def kernel(logit, target):
    n = logit.shape[0]
    tgt = target.astype(jnp.int32).reshape(n, 1)
    out = pl.pallas_call(
        _body,
        grid=(_NBLK,),
        in_specs=[
            pl.BlockSpec((n, 1), lambda j: (0, 0)),
            pl.BlockSpec((n, _BV), lambda j: (0, j)),
        ],
        out_specs=pl.BlockSpec((1, 1), lambda j: (0, 0)),
        out_shape=jax.ShapeDtypeStruct((1, 1), jnp.float32),
        compiler_params=pltpu.CompilerParams(
            dimension_semantics=("arbitrary",),
        ),
    )(tgt, logit)
    return out[0, 0]
